# baseline (speedup 1.0000x reference)
import functools

import jax
import jax.numpy as jnp
from jax import lax
from jax.experimental import pallas as pl
from jax.experimental.pallas import tpu as pltpu
from jax.experimental.pallas import tpu_sc as plsc

N = 50000
E = 800000
IN_CONT = 11
EMB_DIM = 8
HID = 64
HALF = 32

E_PAD = 819200
CH = 128
NROWS = 50176
TILE_ROWS = NROWS // 16

BLK = 2000
GRID = N // BLK
UNROLL = 2
NCHUNKS = E_PAD // CH

_mesh = plsc.VectorSubcoreMesh(core_axis_name="c", subcore_axis_name="s")
_sc_params = pltpu.CompilerParams(use_tc_tiling_on_sc=False)


def _deg_body(dst_hbm, zeros_hbm, ones_hbm, deg0_hbm, deg1_hbm,
              didx, ones_v, deg_sh, sem0, sem1):
    c = lax.axis_index("c")
    s = lax.axis_index("s")
    wid = s * 2 + c
    r0 = s * TILE_ROWS
    pltpu.sync_copy(zeros_hbm.at[pl.ds(r0, TILE_ROWS)],
                    deg_sh.at[pl.ds(r0, TILE_ROWS)])
    pltpu.sync_copy(ones_hbm, ones_v)
    plsc.subcore_barrier()

    tile_chunks = NCHUNKS // 32
    base = wid * tile_chunks
    nhalf = tile_chunks // (2 * UNROLL)
    sems = (sem0, sem1)

    def load_d(st, grp):
        pltpu.sync_copy(dst_hbm.at[pl.ds(base + grp * UNROLL, UNROLL)],
                        didx.at[st])

    def fire_s(st):
        for k in range(UNROLL):
            pltpu.async_copy(ones_v, deg_sh.at[didx.at[st, k]], sems[st],
                             add=True)

    def wait_s(st):
        for k in range(UNROLL):
            pltpu.make_async_copy(ones_v, deg_sh.at[didx.at[st, k]],
                                  sems[st]).wait()

    load_d(0, 0)

    def step(j, carry):
        i = 2 * j
        fire_s(0)

        @pl.when(j > 0)
        def _():
            wait_s(1)

        load_d(1, i + 1)
        fire_s(1)
        wait_s(0)

        @pl.when(j < nhalf - 1)
        def _():
            load_d(0, i + 2)

        return carry

    lax.fori_loop(0, nhalf, step, 0)
    wait_s(1)
    plsc.subcore_barrier()

    @pl.when(c == 0)
    def _():
        pltpu.sync_copy(deg_sh.at[pl.ds(r0, TILE_ROWS)],
                        deg0_hbm.at[pl.ds(r0, TILE_ROWS)])

    @pl.when(c == 1)
    def _():
        pltpu.sync_copy(deg_sh.at[pl.ds(r0, TILE_ROWS)],
                        deg1_hbm.at[pl.ds(r0, TILE_ROWS)])


_deg_call = pl.kernel(
    _deg_body,
    out_type=[jax.ShapeDtypeStruct((NROWS, 16), jnp.float32)] * 2,
    mesh=_mesh,
    scratch_types=[
        pltpu.VMEM((2, UNROLL, CH), jnp.int32),
        pltpu.VMEM((CH, 16), jnp.float32),
        pltpu.VMEM_SHARED((NROWS, 16), jnp.float32),
        pltpu.SemaphoreType.DMA,
        pltpu.SemaphoreType.DMA,
    ],
    compiler_params=_sc_params,
)


def _agg_body(src_hbm, dst_hbm, p0_hbm, p1_hbm, zeros_hbm, out0_hbm, out1_hbm,
              sidx, didx, rows, agg_sh, gsem0, gsem1, ssem0, ssem1):
    c = lax.axis_index("c")
    s = lax.axis_index("s")
    r0 = s * TILE_ROWS
    pltpu.sync_copy(zeros_hbm.at[pl.ds(r0, TILE_ROWS)],
                    agg_sh.at[pl.ds(r0, TILE_ROWS)])
    plsc.subcore_barrier()

    tile_chunks = NCHUNKS // 16
    base = s * tile_chunks
    nhalf = tile_chunks // (2 * UNROLL)
    gsems = (gsem0, gsem1)
    ssems = (ssem0, ssem1)

    def run(p_hbm):
        def load_idx(st, grp):
            j0 = base + grp * UNROLL
            pltpu.sync_copy(src_hbm.at[pl.ds(j0, UNROLL)], sidx.at[st])
            pltpu.sync_copy(dst_hbm.at[pl.ds(j0, UNROLL)], didx.at[st])

        def fire_g(st):
            for k in range(UNROLL):
                pltpu.async_copy(p_hbm.at[sidx.at[st, k]], rows.at[st, k],
                                 gsems[st])

        def wait_g(st):
            for k in range(UNROLL):
                pltpu.make_async_copy(p_hbm.at[sidx.at[st, k]],
                                      rows.at[st, k], gsems[st]).wait()

        def fire_s(st):
            for k in range(UNROLL):
                pltpu.async_copy(rows.at[st, k], agg_sh.at[didx.at[st, k]],
                                 ssems[st], add=True)

        def wait_s(st):
            for k in range(UNROLL):
                pltpu.make_async_copy(rows.at[st, k],
                                      agg_sh.at[didx.at[st, k]],
                                      ssems[st]).wait()

        load_idx(0, 0)
        fire_g(0)

        def step(j, carry):
            i = 2 * j
            wait_g(0)
            fire_s(0)

            @pl.when(j > 0)
            def _():
                wait_s(1)

            load_idx(1, i + 1)
            fire_g(1)
            wait_g(1)
            fire_s(1)
            wait_s(0)

            @pl.when(j < nhalf - 1)
            def _():
                load_idx(0, i + 2)
                fire_g(0)

            return carry

        lax.fori_loop(0, nhalf, step, 0)
        wait_s(1)

    @pl.when(c == 0)
    def _():
        run(p0_hbm)

    @pl.when(c == 1)
    def _():
        run(p1_hbm)

    plsc.subcore_barrier()

    @pl.when(c == 0)
    def _():
        pltpu.sync_copy(agg_sh.at[pl.ds(r0, TILE_ROWS)],
                        out0_hbm.at[pl.ds(r0, TILE_ROWS)])

    @pl.when(c == 1)
    def _():
        pltpu.sync_copy(agg_sh.at[pl.ds(r0, TILE_ROWS)],
                        out1_hbm.at[pl.ds(r0, TILE_ROWS)])


_agg_call = pl.kernel(
    _agg_body,
    out_type=[jax.ShapeDtypeStruct((NROWS, HALF), jnp.float32)] * 2,
    mesh=_mesh,
    scratch_types=[
        pltpu.VMEM((2, UNROLL, CH), jnp.int32),
        pltpu.VMEM((2, UNROLL, CH), jnp.int32),
        pltpu.VMEM((2, UNROLL, CH, HALF), jnp.float32),
        pltpu.VMEM_SHARED((NROWS, HALF), jnp.float32),
        pltpu.SemaphoreType.DMA,
        pltpu.SemaphoreType.DMA,
        pltpu.SemaphoreType.DMA,
        pltpu.SemaphoreType.DMA,
    ],
    compiler_params=_sc_params,
)


def _pre1_kernel(xc_ref, ft_ref, d0_ref, d1_ref, emb_ref, w1a_ref, w1b_ref,
                 p0_ref, p1_ref, h1_ref, dis_ref):
    xc = xc_ref[...]
    t = jnp.dot(emb_ref[...], w1b_ref[...],
                preferred_element_type=jnp.float32)
    h = jnp.dot(xc, w1a_ref[...], preferred_element_type=jnp.float32)
    ft = ft_ref[...]
    h = h + jnp.where(ft == 0, t[0:1, :], t[1:2, :])
    deg = d0_ref[...][:, 0:1] + d1_ref[...][:, 0:1] + 1.0
    dis = lax.rsqrt(deg)
    p = h * dis
    h1_ref[...] = h
    dis_ref[...] = dis
    p0_ref[...] = p[:, :HALF]
    p1_ref[...] = p[:, HALF:]


def _mid_kernel(a0_ref, a1_ref, h1_ref, dis_ref, w2t_ref, b1_ref,
                p0_ref, p1_ref, h2_ref):
    agg = jnp.concatenate([a0_ref[...], a1_ref[...]], axis=1)
    dis = dis_ref[...]
    x = jnp.maximum(agg * dis + h1_ref[...] * (dis * dis) + b1_ref[...], 0.0)
    h2 = jnp.dot(x, w2t_ref[...], preferred_element_type=jnp.float32)
    p = h2 * dis
    h2_ref[...] = h2
    p0_ref[...] = p[:, :HALF]
    p1_ref[...] = p[:, HALF:]


def _fin_kernel(a0_ref, a1_ref, h2_ref, dis_ref, wlt_ref, b2_ref, bl_ref,
                y_ref):
    agg = jnp.concatenate([a0_ref[...], a1_ref[...]], axis=1)
    dis = dis_ref[...]
    x = jnp.maximum(agg * dis + h2_ref[...] * (dis * dis) + b2_ref[...], 0.0)
    y_ref[...] = jnp.dot(x, wlt_ref[...],
                         preferred_element_type=jnp.float32) + bl_ref[...]


def _row_spec(cols):
    return pl.BlockSpec((BLK, cols), lambda i: (i, 0))


def _full_spec(r, c):
    return pl.BlockSpec((r, c), lambda i: (0, 0))


_pre1_call = pl.pallas_call(
    _pre1_kernel,
    grid=(GRID,),
    in_specs=[_row_spec(IN_CONT), _row_spec(1), _row_spec(16), _row_spec(16),
              _full_spec(2, EMB_DIM), _full_spec(IN_CONT, HID),
              _full_spec(EMB_DIM, HID)],
    out_specs=[_row_spec(HALF), _row_spec(HALF), _row_spec(HID), _row_spec(1)],
    out_shape=[jax.ShapeDtypeStruct((N, HALF), jnp.float32),
               jax.ShapeDtypeStruct((N, HALF), jnp.float32),
               jax.ShapeDtypeStruct((N, HID), jnp.float32),
               jax.ShapeDtypeStruct((N, 1), jnp.float32)],
)

_mid_call = pl.pallas_call(
    _mid_kernel,
    grid=(GRID,),
    in_specs=[_row_spec(HALF), _row_spec(HALF), _row_spec(HID), _row_spec(1),
              _full_spec(HID, HID), _full_spec(1, HID)],
    out_specs=[_row_spec(HALF), _row_spec(HALF), _row_spec(HID)],
    out_shape=[jax.ShapeDtypeStruct((N, HALF), jnp.float32),
               jax.ShapeDtypeStruct((N, HALF), jnp.float32),
               jax.ShapeDtypeStruct((N, HID), jnp.float32)],
)

_fin_call = pl.pallas_call(
    _fin_kernel,
    grid=(GRID,),
    in_specs=[_row_spec(HALF), _row_spec(HALF), _row_spec(HID), _row_spec(1),
              _full_spec(HID, 1), _full_spec(1, HID), _full_spec(1, 1)],
    out_specs=_row_spec(1),
    out_shape=jax.ShapeDtypeStruct((N, 1), jnp.float32),
)


def kernel(x_cont, func_type, edge_index, emb, W1, b1, W2, b2, Wlin, blin):
    src = jnp.concatenate([edge_index[0].astype(jnp.int32),
                           jnp.zeros((E_PAD - E,), jnp.int32)])
    dst = jnp.concatenate([edge_index[1].astype(jnp.int32),
                           jnp.full((E_PAD - E,), N, jnp.int32)])
    src = src.reshape(NCHUNKS, CH)
    dst = dst.reshape(NCHUNKS, CH)
    z16 = jnp.zeros((NROWS, 16), jnp.float32)
    z32 = jnp.zeros((NROWS, HALF), jnp.float32)
    ones_ch = jnp.ones((CH, 16), jnp.float32)

    deg0, deg1 = _deg_call(dst, z16, ones_ch)

    ft2 = func_type.astype(jnp.int32).reshape(N, 1)
    w1a = W1[:, :IN_CONT].T
    w1b = W1[:, IN_CONT:].T
    p0, p1, h1, dis = _pre1_call(x_cont, ft2, deg0, deg1, emb, w1a, w1b)

    a0, a1 = _agg_call(src, dst, p0, p1, z32)
    q0, q1, h2 = _mid_call(a0, a1, h1, dis, W2.T, b1.reshape(1, HID))
    g0, g1 = _agg_call(src, dst, q0, q1, z32)
    y = _fin_call(g0, g1, h2, dis, Wlin.T, b2.reshape(1, HID),
                  blin.reshape(1, 1))
    return y.reshape(N)


# speedup vs baseline: 1.1934x; 1.1934x over previous
import functools

import jax
import jax.numpy as jnp
from jax import lax
from jax.experimental import pallas as pl
from jax.experimental.pallas import tpu as pltpu
from jax.experimental.pallas import tpu_sc as plsc

N = 50000
E = 800000
IN_CONT = 11
EMB_DIM = 8
HID = 64
HALF = 32

CH = 112
E_PAD = 817152
NROWS = 50176
TILE_ROWS = NROWS // 16

BLK = 2000
GRID = N // BLK
UNROLL = 4
DEG_UNROLL = 2
NCHUNKS = E_PAD // CH

_mesh = plsc.VectorSubcoreMesh(core_axis_name="c", subcore_axis_name="s")
_sc_params = pltpu.CompilerParams(use_tc_tiling_on_sc=False)


def _deg_body(dst_hbm, zeros_hbm, ones_hbm, deg0_hbm, deg1_hbm,
              didx, ones_v, deg_sh, sem0, sem1):
    c = lax.axis_index("c")
    s = lax.axis_index("s")
    wid = s * 2 + c
    r0 = s * TILE_ROWS
    pltpu.sync_copy(zeros_hbm.at[pl.ds(r0, TILE_ROWS)],
                    deg_sh.at[pl.ds(r0, TILE_ROWS)])
    pltpu.sync_copy(ones_hbm, ones_v)
    plsc.subcore_barrier()

    tile_chunks = NCHUNKS // 32
    base = wid * tile_chunks
    nhalf = tile_chunks // (2 * DEG_UNROLL)
    sems = (sem0, sem1)

    def load_d(st, grp):
        pltpu.sync_copy(dst_hbm.at[pl.ds(base + grp * DEG_UNROLL, DEG_UNROLL)],
                        didx.at[st])

    def fire_s(st):
        for k in range(DEG_UNROLL):
            pltpu.async_copy(ones_v, deg_sh.at[didx.at[st, k]], sems[st],
                             add=True)

    def wait_s(st):
        for k in range(DEG_UNROLL):
            pltpu.make_async_copy(ones_v, deg_sh.at[didx.at[st, k]],
                                  sems[st]).wait()

    load_d(0, 0)

    def step(j, carry):
        i = 2 * j
        fire_s(0)

        @pl.when(j > 0)
        def _():
            wait_s(1)

        load_d(1, i + 1)
        fire_s(1)
        wait_s(0)

        @pl.when(j < nhalf - 1)
        def _():
            load_d(0, i + 2)

        return carry

    lax.fori_loop(0, nhalf, step, 0)
    wait_s(1)
    plsc.subcore_barrier()

    @pl.when(c == 0)
    def _():
        pltpu.sync_copy(deg_sh.at[pl.ds(r0, TILE_ROWS)],
                        deg0_hbm.at[pl.ds(r0, TILE_ROWS)])

    @pl.when(c == 1)
    def _():
        pltpu.sync_copy(deg_sh.at[pl.ds(r0, TILE_ROWS)],
                        deg1_hbm.at[pl.ds(r0, TILE_ROWS)])


_deg_call = pl.kernel(
    _deg_body,
    out_type=[jax.ShapeDtypeStruct((NROWS, 16), jnp.float32)] * 2,
    mesh=_mesh,
    scratch_types=[
        pltpu.VMEM((2, DEG_UNROLL, CH), jnp.int32),
        pltpu.VMEM((CH, 16), jnp.float32),
        pltpu.VMEM_SHARED((NROWS, 16), jnp.float32),
        pltpu.SemaphoreType.DMA,
        pltpu.SemaphoreType.DMA,
    ],
    compiler_params=_sc_params,
)


def _agg_body(src_hbm, dst_hbm, p0_hbm, p1_hbm, zeros_hbm, out0_hbm, out1_hbm,
              sidx, didx, rows, agg_sh, gsem0, gsem1, ssem0, ssem1):
    c = lax.axis_index("c")
    s = lax.axis_index("s")
    r0 = s * TILE_ROWS
    pltpu.sync_copy(zeros_hbm.at[pl.ds(r0, TILE_ROWS)],
                    agg_sh.at[pl.ds(r0, TILE_ROWS)])
    plsc.subcore_barrier()

    tile_chunks = NCHUNKS // 16
    base = s * tile_chunks
    nhalf = tile_chunks // (2 * UNROLL)
    gsems = (gsem0, gsem1)
    ssems = (ssem0, ssem1)

    def run(p_hbm):
        def load_idx(st, grp):
            j0 = base + grp * UNROLL
            pltpu.sync_copy(src_hbm.at[pl.ds(j0, UNROLL)], sidx.at[st])
            pltpu.sync_copy(dst_hbm.at[pl.ds(j0, UNROLL)], didx.at[st])

        def fire_g(st):
            for k in range(UNROLL):
                pltpu.async_copy(p_hbm.at[sidx.at[st, k]], rows.at[st, k],
                                 gsems[st])

        def wait_g(st):
            for k in range(UNROLL):
                pltpu.make_async_copy(p_hbm.at[sidx.at[st, k]],
                                      rows.at[st, k], gsems[st]).wait()

        def fire_s(st):
            for k in range(UNROLL):
                pltpu.async_copy(rows.at[st, k], agg_sh.at[didx.at[st, k]],
                                 ssems[st], add=True)

        def wait_s(st):
            for k in range(UNROLL):
                pltpu.make_async_copy(rows.at[st, k],
                                      agg_sh.at[didx.at[st, k]],
                                      ssems[st]).wait()

        load_idx(0, 0)
        fire_g(0)

        def step(j, carry):
            i = 2 * j
            wait_g(0)
            fire_s(0)

            @pl.when(j > 0)
            def _():
                wait_s(1)

            load_idx(1, i + 1)
            fire_g(1)
            wait_g(1)
            fire_s(1)
            wait_s(0)

            @pl.when(j < nhalf - 1)
            def _():
                load_idx(0, i + 2)
                fire_g(0)

            return carry

        lax.fori_loop(0, nhalf, step, 0)
        wait_s(1)

    @pl.when(c == 0)
    def _():
        run(p0_hbm)

    @pl.when(c == 1)
    def _():
        run(p1_hbm)

    plsc.subcore_barrier()

    @pl.when(c == 0)
    def _():
        pltpu.sync_copy(agg_sh.at[pl.ds(r0, TILE_ROWS)],
                        out0_hbm.at[pl.ds(r0, TILE_ROWS)])

    @pl.when(c == 1)
    def _():
        pltpu.sync_copy(agg_sh.at[pl.ds(r0, TILE_ROWS)],
                        out1_hbm.at[pl.ds(r0, TILE_ROWS)])


_agg_call = pl.kernel(
    _agg_body,
    out_type=[jax.ShapeDtypeStruct((NROWS, HALF), jnp.float32)] * 2,
    mesh=_mesh,
    scratch_types=[
        pltpu.VMEM((2, UNROLL, CH), jnp.int32),
        pltpu.VMEM((2, UNROLL, CH), jnp.int32),
        pltpu.VMEM((2, UNROLL, CH, HALF), jnp.float32),
        pltpu.VMEM_SHARED((NROWS, HALF), jnp.float32),
        pltpu.SemaphoreType.DMA,
        pltpu.SemaphoreType.DMA,
        pltpu.SemaphoreType.DMA,
        pltpu.SemaphoreType.DMA,
    ],
    compiler_params=_sc_params,
)


def _pre1_kernel(xc_ref, ft_ref, d0_ref, d1_ref, emb_ref, w1a_ref, w1b_ref,
                 p0_ref, p1_ref, h1_ref, dis_ref):
    xc = xc_ref[...]
    t = jnp.dot(emb_ref[...], w1b_ref[...],
                preferred_element_type=jnp.float32)
    h = jnp.dot(xc, w1a_ref[...], preferred_element_type=jnp.float32)
    ft = ft_ref[...]
    h = h + jnp.where(ft == 0, t[0:1, :], t[1:2, :])
    deg = d0_ref[...][:, 0:1] + d1_ref[...][:, 0:1] + 1.0
    dis = lax.rsqrt(deg)
    p = h * dis
    h1_ref[...] = h
    dis_ref[...] = dis
    p0_ref[...] = p[:, :HALF]
    p1_ref[...] = p[:, HALF:]


def _mid_kernel(a0_ref, a1_ref, h1_ref, dis_ref, w2t_ref, b1_ref,
                p0_ref, p1_ref, h2_ref):
    agg = jnp.concatenate([a0_ref[...], a1_ref[...]], axis=1)
    dis = dis_ref[...]
    x = jnp.maximum(agg * dis + h1_ref[...] * (dis * dis) + b1_ref[...], 0.0)
    h2 = jnp.dot(x, w2t_ref[...], preferred_element_type=jnp.float32)
    p = h2 * dis
    h2_ref[...] = h2
    p0_ref[...] = p[:, :HALF]
    p1_ref[...] = p[:, HALF:]


def _fin_kernel(a0_ref, a1_ref, h2_ref, dis_ref, wlt_ref, b2_ref, bl_ref,
                y_ref):
    agg = jnp.concatenate([a0_ref[...], a1_ref[...]], axis=1)
    dis = dis_ref[...]
    x = jnp.maximum(agg * dis + h2_ref[...] * (dis * dis) + b2_ref[...], 0.0)
    y_ref[...] = jnp.dot(x, wlt_ref[...],
                         preferred_element_type=jnp.float32) + bl_ref[...]


def _row_spec(cols):
    return pl.BlockSpec((BLK, cols), lambda i: (i, 0))


def _full_spec(r, c):
    return pl.BlockSpec((r, c), lambda i: (0, 0))


_pre1_call = pl.pallas_call(
    _pre1_kernel,
    grid=(GRID,),
    in_specs=[_row_spec(IN_CONT), _row_spec(1), _row_spec(16), _row_spec(16),
              _full_spec(2, EMB_DIM), _full_spec(IN_CONT, HID),
              _full_spec(EMB_DIM, HID)],
    out_specs=[_row_spec(HALF), _row_spec(HALF), _row_spec(HID), _row_spec(1)],
    out_shape=[jax.ShapeDtypeStruct((N, HALF), jnp.float32),
               jax.ShapeDtypeStruct((N, HALF), jnp.float32),
               jax.ShapeDtypeStruct((N, HID), jnp.float32),
               jax.ShapeDtypeStruct((N, 1), jnp.float32)],
)

_mid_call = pl.pallas_call(
    _mid_kernel,
    grid=(GRID,),
    in_specs=[_row_spec(HALF), _row_spec(HALF), _row_spec(HID), _row_spec(1),
              _full_spec(HID, HID), _full_spec(1, HID)],
    out_specs=[_row_spec(HALF), _row_spec(HALF), _row_spec(HID)],
    out_shape=[jax.ShapeDtypeStruct((N, HALF), jnp.float32),
               jax.ShapeDtypeStruct((N, HALF), jnp.float32),
               jax.ShapeDtypeStruct((N, HID), jnp.float32)],
)

_fin_call = pl.pallas_call(
    _fin_kernel,
    grid=(GRID,),
    in_specs=[_row_spec(HALF), _row_spec(HALF), _row_spec(HID), _row_spec(1),
              _full_spec(HID, 1), _full_spec(1, HID), _full_spec(1, 1)],
    out_specs=_row_spec(1),
    out_shape=jax.ShapeDtypeStruct((N, 1), jnp.float32),
)


def kernel(x_cont, func_type, edge_index, emb, W1, b1, W2, b2, Wlin, blin):
    src = jnp.concatenate([edge_index[0].astype(jnp.int32),
                           jnp.zeros((E_PAD - E,), jnp.int32)])
    dst = jnp.concatenate([edge_index[1].astype(jnp.int32),
                           jnp.full((E_PAD - E,), N, jnp.int32)])
    src = src.reshape(NCHUNKS, CH)
    dst = dst.reshape(NCHUNKS, CH)
    z16 = jnp.zeros((NROWS, 16), jnp.float32)
    z32 = jnp.zeros((NROWS, HALF), jnp.float32)
    ones_ch = jnp.ones((CH, 16), jnp.float32)

    deg0, deg1 = _deg_call(dst, z16, ones_ch)

    ft2 = func_type.astype(jnp.int32).reshape(N, 1)
    w1a = W1[:, :IN_CONT].T
    w1b = W1[:, IN_CONT:].T
    p0, p1, h1, dis = _pre1_call(x_cont, ft2, deg0, deg1, emb, w1a, w1b)

    a0, a1 = _agg_call(src, dst, p0, p1, z32)
    q0, q1, h2 = _mid_call(a0, a1, h1, dis, W2.T, b1.reshape(1, HID))
    g0, g1 = _agg_call(src, dst, q0, q1, z32)
    y = _fin_call(g0, g1, h2, dis, Wlin.T, b2.reshape(1, HID),
                  blin.reshape(1, 1))
    return y.reshape(N)


# speedup vs baseline: 1.4248x; 1.1938x over previous
import functools

import numpy as np
import jax
import jax.numpy as jnp
from jax import lax
from jax.experimental import pallas as pl
from jax.experimental.pallas import tpu as pltpu
from jax.experimental.pallas import tpu_sc as plsc

N = 50000
E = 800000
IN_CONT = 11
EMB_DIM = 8
HID = 64
HALF = 32

CH = 112
E_PAD = 817152
NROWS = 50176
TILE_ROWS = NROWS // 16

BLK = 2000
GRID = N // BLK
UNROLL = 2
DEG_UNROLL = 2
NCHUNKS = E_PAD // CH

_mesh = plsc.VectorSubcoreMesh(core_axis_name="c", subcore_axis_name="s")

_Z16 = np.zeros((50176, 16), np.float32)
_Z32 = np.zeros((50176, 32), np.float32)
_ONES = np.ones((112, 16), np.float32)
_sc_params = pltpu.CompilerParams(use_tc_tiling_on_sc=False)


def _deg_body(dst_hbm, zeros_hbm, ones_hbm, deg0_hbm, deg1_hbm,
              didx, ones_v, deg_sh, sem0, sem1):
    c = lax.axis_index("c")
    s = lax.axis_index("s")
    wid = s * 2 + c
    r0 = s * TILE_ROWS
    pltpu.sync_copy(zeros_hbm.at[pl.ds(r0, TILE_ROWS)],
                    deg_sh.at[pl.ds(r0, TILE_ROWS)])
    pltpu.sync_copy(ones_hbm, ones_v)
    plsc.subcore_barrier()

    tile_chunks = NCHUNKS // 32
    base = wid * tile_chunks
    nhalf = tile_chunks // (2 * DEG_UNROLL)
    sems = (sem0, sem1)

    def load_d(st, grp):
        pltpu.sync_copy(dst_hbm.at[pl.ds(base + grp * DEG_UNROLL, DEG_UNROLL)],
                        didx.at[st])

    def fire_s(st):
        for k in range(DEG_UNROLL):
            pltpu.async_copy(ones_v, deg_sh.at[didx.at[st, k]], sems[st],
                             add=True)

    def wait_s(st):
        for k in range(DEG_UNROLL):
            pltpu.make_async_copy(ones_v, deg_sh.at[didx.at[st, k]],
                                  sems[st]).wait()

    load_d(0, 0)

    def step(j, carry):
        i = 2 * j
        fire_s(0)

        @pl.when(j > 0)
        def _():
            wait_s(1)

        load_d(1, i + 1)
        fire_s(1)
        wait_s(0)

        @pl.when(j < nhalf - 1)
        def _():
            load_d(0, i + 2)

        return carry

    lax.fori_loop(0, nhalf, step, 0)
    wait_s(1)
    plsc.subcore_barrier()

    @pl.when(c == 0)
    def _():
        pltpu.sync_copy(deg_sh.at[pl.ds(r0, TILE_ROWS)],
                        deg0_hbm.at[pl.ds(r0, TILE_ROWS)])

    @pl.when(c == 1)
    def _():
        pltpu.sync_copy(deg_sh.at[pl.ds(r0, TILE_ROWS)],
                        deg1_hbm.at[pl.ds(r0, TILE_ROWS)])


_deg_call = pl.kernel(
    _deg_body,
    out_type=[jax.ShapeDtypeStruct((NROWS, 16), jnp.float32)] * 2,
    mesh=_mesh,
    scratch_types=[
        pltpu.VMEM((2, DEG_UNROLL, CH), jnp.int32),
        pltpu.VMEM((CH, 16), jnp.float32),
        pltpu.VMEM_SHARED((NROWS, 16), jnp.float32),
        pltpu.SemaphoreType.DMA,
        pltpu.SemaphoreType.DMA,
    ],
    compiler_params=_sc_params,
)


def _agg_body(src_hbm, dst_hbm, p0_hbm, p1_hbm, zeros_hbm, out0_hbm, out1_hbm,
              sidx, didx, rows, agg_sh,
              isem0, isem1, isem2, isem3, gsem0, gsem1, gsem2,
              ssem0, ssem1, ssem2):
    c = lax.axis_index("c")
    s = lax.axis_index("s")
    r0 = s * TILE_ROWS
    pltpu.sync_copy(zeros_hbm.at[pl.ds(r0, TILE_ROWS)],
                    agg_sh.at[pl.ds(r0, TILE_ROWS)])
    plsc.subcore_barrier()

    tile_chunks = NCHUNKS // 16
    base = s * tile_chunks
    ng = tile_chunks // UNROLL
    isems = (isem0, isem1, isem2, isem3)
    gsems = (gsem0, gsem1, gsem2)
    ssems = (ssem0, ssem1, ssem2)

    def run(p_hbm):
        def load_idx(x, grp):
            j0 = base + grp * UNROLL
            pltpu.async_copy(src_hbm.at[pl.ds(j0, UNROLL)], sidx.at[x],
                             isems[x])
            pltpu.async_copy(dst_hbm.at[pl.ds(j0, UNROLL)], didx.at[x],
                             isems[x])

        def wait_idx(x):
            pltpu.make_async_copy(src_hbm.at[pl.ds(base, UNROLL)],
                                  sidx.at[x], isems[x]).wait()
            pltpu.make_async_copy(dst_hbm.at[pl.ds(base, UNROLL)],
                                  didx.at[x], isems[x]).wait()

        def fire_g(r, x):
            for u in range(UNROLL):
                pltpu.async_copy(p_hbm.at[sidx.at[x, u]], rows.at[r, u],
                                 gsems[r])

        def wait_g(r, x):
            for u in range(UNROLL):
                pltpu.make_async_copy(p_hbm.at[sidx.at[x, u]],
                                      rows.at[r, u], gsems[r]).wait()

        def fire_s(r, x):
            for u in range(UNROLL):
                pltpu.async_copy(rows.at[r, u], agg_sh.at[didx.at[x, u]],
                                 ssems[r], add=True)

        def wait_s(r, x):
            for u in range(UNROLL):
                pltpu.make_async_copy(rows.at[r, u],
                                      agg_sh.at[didx.at[x, u]],
                                      ssems[r]).wait()

        load_idx(0, 0)
        load_idx(1, 1)
        wait_idx(0)
        fire_g(0, 0)

        def step(m, carry):
            for k in range(12):
                g = 12 * m + k
                r, x = k % 3, k % 4
                r1, x1 = (k + 1) % 3, (k + 1) % 4
                x2 = (k + 2) % 4
                rp, xp = (k - 2) % 3, (k - 2) % 4

                @pl.when(g + 1 < ng)
                def _():
                    wait_idx(x1)

                @pl.when(g >= 2)
                def _():
                    wait_s(rp, xp)

                @pl.when(g + 1 < ng)
                def _():
                    fire_g(r1, x1)

                @pl.when(g + 2 < ng)
                def _():
                    load_idx(x2, g + 2)

                wait_g(r, x)
                fire_s(r, x)
            return carry

        lax.fori_loop(0, ng // 12, step, 0)
        wait_s((ng - 2) % 3, (ng - 2) % 4)
        wait_s((ng - 1) % 3, (ng - 1) % 4)

    @pl.when(c == 0)
    def _():
        run(p0_hbm)

    @pl.when(c == 1)
    def _():
        run(p1_hbm)

    plsc.subcore_barrier()

    @pl.when(c == 0)
    def _():
        pltpu.sync_copy(agg_sh.at[pl.ds(r0, TILE_ROWS)],
                        out0_hbm.at[pl.ds(r0, TILE_ROWS)])

    @pl.when(c == 1)
    def _():
        pltpu.sync_copy(agg_sh.at[pl.ds(r0, TILE_ROWS)],
                        out1_hbm.at[pl.ds(r0, TILE_ROWS)])


_agg_call = pl.kernel(
    _agg_body,
    out_type=[jax.ShapeDtypeStruct((NROWS, HALF), jnp.float32)] * 2,
    mesh=_mesh,
    scratch_types=[
        pltpu.VMEM((4, UNROLL, CH), jnp.int32),
        pltpu.VMEM((4, UNROLL, CH), jnp.int32),
        pltpu.VMEM((3, UNROLL, CH, HALF), jnp.float32),
        pltpu.VMEM_SHARED((NROWS, HALF), jnp.float32),
    ] + [pltpu.SemaphoreType.DMA] * 10,
    compiler_params=_sc_params,
)


def _pre1_kernel(xc_ref, ft_ref, d0_ref, d1_ref, emb_ref, w1a_ref, w1b_ref,
                 p0_ref, p1_ref, h1_ref, dis_ref):
    xc = xc_ref[...]
    t = jnp.dot(emb_ref[...], w1b_ref[...],
                preferred_element_type=jnp.float32)
    h = jnp.dot(xc, w1a_ref[...], preferred_element_type=jnp.float32)
    ft = ft_ref[...]
    h = h + jnp.where(ft == 0, t[0:1, :], t[1:2, :])
    deg = d0_ref[...][:, 0:1] + d1_ref[...][:, 0:1] + 1.0
    dis = lax.rsqrt(deg)
    p = h * dis
    h1_ref[...] = h
    dis_ref[...] = dis
    p0_ref[...] = p[:, :HALF]
    p1_ref[...] = p[:, HALF:]


def _mid_kernel(a0_ref, a1_ref, h1_ref, dis_ref, w2t_ref, b1_ref,
                p0_ref, p1_ref, h2_ref):
    agg = jnp.concatenate([a0_ref[...], a1_ref[...]], axis=1)
    dis = dis_ref[...]
    x = jnp.maximum(agg * dis + h1_ref[...] * (dis * dis) + b1_ref[...], 0.0)
    h2 = jnp.dot(x, w2t_ref[...], preferred_element_type=jnp.float32)
    p = h2 * dis
    h2_ref[...] = h2
    p0_ref[...] = p[:, :HALF]
    p1_ref[...] = p[:, HALF:]


def _fin_kernel(a0_ref, a1_ref, h2_ref, dis_ref, wlt_ref, b2_ref, bl_ref,
                y_ref):
    agg = jnp.concatenate([a0_ref[...], a1_ref[...]], axis=1)
    dis = dis_ref[...]
    x = jnp.maximum(agg * dis + h2_ref[...] * (dis * dis) + b2_ref[...], 0.0)
    y_ref[...] = jnp.dot(x, wlt_ref[...],
                         preferred_element_type=jnp.float32) + bl_ref[...]


def _row_spec(cols):
    return pl.BlockSpec((BLK, cols), lambda i: (i, 0))


def _full_spec(r, c):
    return pl.BlockSpec((r, c), lambda i: (0, 0))


_pre1_call = pl.pallas_call(
    _pre1_kernel,
    grid=(GRID,),
    in_specs=[_row_spec(IN_CONT), _row_spec(1), _row_spec(16), _row_spec(16),
              _full_spec(2, EMB_DIM), _full_spec(IN_CONT, HID),
              _full_spec(EMB_DIM, HID)],
    out_specs=[_row_spec(HALF), _row_spec(HALF), _row_spec(HID), _row_spec(1)],
    out_shape=[jax.ShapeDtypeStruct((N, HALF), jnp.float32),
               jax.ShapeDtypeStruct((N, HALF), jnp.float32),
               jax.ShapeDtypeStruct((N, HID), jnp.float32),
               jax.ShapeDtypeStruct((N, 1), jnp.float32)],
)

_mid_call = pl.pallas_call(
    _mid_kernel,
    grid=(GRID,),
    in_specs=[_row_spec(HALF), _row_spec(HALF), _row_spec(HID), _row_spec(1),
              _full_spec(HID, HID), _full_spec(1, HID)],
    out_specs=[_row_spec(HALF), _row_spec(HALF), _row_spec(HID)],
    out_shape=[jax.ShapeDtypeStruct((N, HALF), jnp.float32),
               jax.ShapeDtypeStruct((N, HALF), jnp.float32),
               jax.ShapeDtypeStruct((N, HID), jnp.float32)],
)

_fin_call = pl.pallas_call(
    _fin_kernel,
    grid=(GRID,),
    in_specs=[_row_spec(HALF), _row_spec(HALF), _row_spec(HID), _row_spec(1),
              _full_spec(HID, 1), _full_spec(1, HID), _full_spec(1, 1)],
    out_specs=_row_spec(1),
    out_shape=jax.ShapeDtypeStruct((N, 1), jnp.float32),
)


def kernel(x_cont, func_type, edge_index, emb, W1, b1, W2, b2, Wlin, blin):
    src = jnp.concatenate([edge_index[0].astype(jnp.int32),
                           jnp.zeros((E_PAD - E,), jnp.int32)])
    dst = jnp.concatenate([edge_index[1].astype(jnp.int32),
                           jnp.full((E_PAD - E,), N, jnp.int32)])
    src = src.reshape(NCHUNKS, CH)
    dst = dst.reshape(NCHUNKS, CH)
    deg0, deg1 = _deg_call(dst, _Z16, _ONES)

    ft2 = func_type.astype(jnp.int32).reshape(N, 1)
    w1a = W1[:, :IN_CONT].T
    w1b = W1[:, IN_CONT:].T
    p0, p1, h1, dis = _pre1_call(x_cont, ft2, deg0, deg1, emb, w1a, w1b)

    a0, a1 = _agg_call(src, dst, p0, p1, _Z32)
    q0, q1, h2 = _mid_call(a0, a1, h1, dis, W2.T, b1.reshape(1, HID))
    g0, g1 = _agg_call(src, dst, q0, q1, _Z32)
    y = _fin_call(g0, g1, h2, dis, Wlin.T, b2.reshape(1, HID),
                  blin.reshape(1, 1))
    return y.reshape(N)
